# baseline (speedup 1.0000x reference)
import functools

import jax
import jax.numpy as jnp
from jax import lax
from jax.experimental import pallas as pl
from jax.experimental.pallas import tpu as pltpu
from jax.experimental.pallas import tpu_sc as plsc

B = 1024
C_IN = 26
C_OUT = 36
NM = 1024
NC, NS, L = 2, 16, 16
NW = NC * NS
B_PER_W = B // NW
CHUNK = 2
NCHUNK = B_PER_W // CHUNK
NVEC = NM // L


def _sc_body(x_hbm, age_hbm, gen_hbm, occ_hbm, out_hbm,
             age_v, gen_v, occ_v, in_buf, emb_buf):
    c = lax.axis_index("c")
    s = lax.axis_index("s")
    wid = s * NC + c
    base = wid * B_PER_W

    pltpu.sync_copy(age_hbm, age_v)
    pltpu.sync_copy(gen_hbm, gen_v)
    pltpu.sync_copy(occ_hbm, occ_v)

    def per_chunk(g, carry):
        m = base + g * CHUNK
        ms = pl.ds(m, CHUNK)
        pltpu.sync_copy(x_hbm.at[ms], in_buf)

        for k in range(CHUNK):
            def per_vec(v, carry2, k=k):
                sl = pl.ds(v * L, L)
                ai = in_buf[k, 21, sl].astype(jnp.int32)
                gi = in_buf[k, 22, sl].astype(jnp.int32)
                oi = in_buf[k, 23, sl].astype(jnp.int32)
                for d in range(4):
                    col = jnp.full((L,), d, jnp.int32)
                    emb_buf[k, d, sl] = plsc.load_gather(age_v, [ai, col])
                for d in range(2):
                    col = jnp.full((L,), d, jnp.int32)
                    emb_buf[k, 4 + d, sl] = plsc.load_gather(gen_v, [gi, col])
                for d in range(8):
                    col = jnp.full((L,), d, jnp.int32)
                    emb_buf[k, 6 + d, sl] = plsc.load_gather(occ_v, [oi, col])
                return carry2

            lax.fori_loop(0, NVEC, per_vec, 0)

        pltpu.sync_copy(in_buf.at[:, pl.ds(0, 18)], out_hbm.at[ms, pl.ds(0, 18)])
        pltpu.sync_copy(in_buf.at[:, pl.ds(19, 2)], out_hbm.at[ms, pl.ds(18, 2)])
        pltpu.sync_copy(in_buf.at[:, pl.ds(24, 2)], out_hbm.at[ms, pl.ds(34, 2)])
        pltpu.sync_copy(emb_buf, out_hbm.at[ms, pl.ds(20, 14)])
        return carry

    lax.fori_loop(0, NCHUNK, per_chunk, 0)


@jax.jit
def kernel(x, age_table, gender_table, occupation_table):
    x3 = x.reshape(B, C_IN, NM)
    mesh = plsc.VectorSubcoreMesh(core_axis_name="c", subcore_axis_name="s",
                                  num_cores=NC, num_subcores=NS)
    out = pl.kernel(
        _sc_body,
        out_type=jax.ShapeDtypeStruct((B, C_OUT, NM), jnp.float32),
        mesh=mesh,
        scratch_types=[
            pltpu.VMEM((7, 4), jnp.float32),
            pltpu.VMEM((2, 2), jnp.float32),
            pltpu.VMEM((21, 8), jnp.float32),
            pltpu.VMEM((CHUNK, C_IN, NM), jnp.float32),
            pltpu.VMEM((CHUNK, 14, NM), jnp.float32),
        ],
        compiler_params=pltpu.CompilerParams(use_tc_tiling_on_sc=False,
                                             needs_layout_passes=False),
    )(x3, age_table, gender_table, occupation_table)
    return out.reshape(B, C_OUT, 32, 32)


# speedup vs baseline: 1.1804x; 1.1804x over previous
import functools

import jax
import jax.numpy as jnp
from jax import lax
from jax.experimental import pallas as pl
from jax.experimental.pallas import tpu as pltpu
from jax.experimental.pallas import tpu_sc as plsc

B = 1024
C_IN = 26
C_OUT = 36
NM = 1024
NC, NS, L = 2, 16, 16
NW = NC * NS
B_PER_W = B // NW
NVEC = NM // L
NBUF = 3
GEN_OFF = 28.0
OCC_OFF = 32.0
CTAB = 200


def _fire_out(in_buf, emb_buf, out_hbm, k, m, sem):
    copies = (
        pltpu.make_async_copy(in_buf.at[k, pl.ds(0, 18)],
                              out_hbm.at[m, pl.ds(0, 18)], sem),
        pltpu.make_async_copy(in_buf.at[k, pl.ds(19, 2)],
                              out_hbm.at[m, pl.ds(18, 2)], sem),
        pltpu.make_async_copy(in_buf.at[k, pl.ds(24, 2)],
                              out_hbm.at[m, pl.ds(34, 2)], sem),
        pltpu.make_async_copy(emb_buf.at[k], out_hbm.at[m, pl.ds(20, 14)], sem),
    )
    for cp in copies:
        cp.start()
    return copies


def _drain_out(in_buf, emb_buf, out_hbm, k, m, sem):
    pltpu.make_async_copy(in_buf.at[k, pl.ds(0, 18)],
                          out_hbm.at[m, pl.ds(0, 18)], sem).wait()
    pltpu.make_async_copy(in_buf.at[k, pl.ds(19, 2)],
                          out_hbm.at[m, pl.ds(18, 2)], sem).wait()
    pltpu.make_async_copy(in_buf.at[k, pl.ds(24, 2)],
                          out_hbm.at[m, pl.ds(34, 2)], sem).wait()
    pltpu.make_async_copy(emb_buf.at[k], out_hbm.at[m, pl.ds(20, 14)], sem).wait()


def _sc_body(x_hbm, ctab_hbm, out_hbm, ctab_v, in_buf, emb_buf,
             si0, si1, si2, so0, so1, so2):
    si = (si0, si1, si2)
    so = (so0, so1, so2)
    c = lax.axis_index("c")
    s = lax.axis_index("s")
    wid = s * NC + c
    base = wid * B_PER_W

    pltpu.sync_copy(ctab_hbm, ctab_v)

    pltpu.async_copy(x_hbm.at[base], in_buf.at[0], si[0])

    def compute(k):
        def per_vec(v, carry):
            sl = pl.ds(v * L, L)
            av = in_buf[k, 21, sl]
            gv = in_buf[k, 22, sl]
            ov = in_buf[k, 23, sl]
            ab = (av * 4.0).astype(jnp.int32)
            gb = (gv * 2.0 + GEN_OFF).astype(jnp.int32)
            ob = (ov * 8.0 + OCC_OFF).astype(jnp.int32)
            emb_buf[k, 0, sl] = plsc.load_gather(ctab_v, [ab])
            for d in range(1, 4):
                emb_buf[k, d, sl] = plsc.load_gather(ctab_v, [ab + d])
            emb_buf[k, 4, sl] = plsc.load_gather(ctab_v, [gb])
            emb_buf[k, 5, sl] = plsc.load_gather(ctab_v, [gb + 1])
            emb_buf[k, 6, sl] = plsc.load_gather(ctab_v, [ob])
            for d in range(1, 8):
                emb_buf[k, 6 + d, sl] = plsc.load_gather(ctab_v, [ob + d])
            return carry

        lax.fori_loop(0, NVEC, per_vec, 0, unroll=4)

    def per_turn(g, carry):
        for k in range(NBUF):
            i = g * NBUF + k
            m = base + i

            k2 = (k + 1) % NBUF

            @pl.when(i >= 2)
            def _():
                _drain_out(in_buf, emb_buf, out_hbm, k2, m - 2, so[k2])

            @pl.when(i + 1 < B_PER_W)
            def _():
                pltpu.async_copy(x_hbm.at[m + 1], in_buf.at[k2], si[k2])

            @pl.when(i < B_PER_W)
            def _():
                pltpu.make_async_copy(x_hbm.at[m], in_buf.at[k], si[k]).wait()
                compute(k)
                _fire_out(in_buf, emb_buf, out_hbm, k, m, so[k])

        return carry

    lax.fori_loop(0, (B_PER_W + NBUF) // NBUF, per_turn, 0)

    _drain_out(in_buf, emb_buf, out_hbm, 1, base + B_PER_W - 1, so[1])


@jax.jit
def kernel(x, age_table, gender_table, occupation_table):
    x3 = x.reshape(B, C_IN, NM)
    ctab = jnp.concatenate([age_table.reshape(-1), gender_table.reshape(-1),
                            occupation_table.reshape(-1)])
    mesh = plsc.VectorSubcoreMesh(core_axis_name="c", subcore_axis_name="s",
                                  num_cores=NC, num_subcores=NS)
    out = pl.kernel(
        _sc_body,
        out_type=jax.ShapeDtypeStruct((B, C_OUT, NM), jnp.float32),
        mesh=mesh,
        scratch_types=[
            pltpu.VMEM((CTAB,), jnp.float32),
            pltpu.VMEM((NBUF, C_IN, NM), jnp.float32),
            pltpu.VMEM((NBUF, 14, NM), jnp.float32),
            pltpu.SemaphoreType.DMA,
            pltpu.SemaphoreType.DMA,
            pltpu.SemaphoreType.DMA,
            pltpu.SemaphoreType.DMA,
            pltpu.SemaphoreType.DMA,
            pltpu.SemaphoreType.DMA,
        ],
        compiler_params=pltpu.CompilerParams(use_tc_tiling_on_sc=False,
                                             needs_layout_passes=False),
    )(x3, ctab)
    return out.reshape(B, C_OUT, 32, 32)
